# baseline (speedup 1.0000x reference)
import functools

import jax
import jax.numpy as jnp
from jax import lax
from jax.experimental import pallas as pl
from jax.experimental.pallas import tpu as pltpu
from jax.experimental.pallas import tpu_sc as plsc

NC = 2
NS = 16
L = 16
NW = NC * NS

POS_IOU = 0.7
NEG_IOU = 0.3


def _vlog(x):
    bits = plsc.bitcast(x, jnp.int32)
    e = ((bits >> 23) & 0xFF) - 127
    m = plsc.bitcast((bits & 0x007FFFFF) | 0x3F800000, jnp.float32)
    half = m * 0.5
    big = half >= 0.70710678118654752440
    xr = jnp.where(big, half, m) - 1.0
    e = (e + jnp.where(big, 1, 0)).astype(jnp.float32)
    z = xr * xr
    p = jnp.full_like(xr, 7.0376836292e-2)
    for c in (-1.1514610310e-1, 1.1676998740e-1, -1.2420140846e-1,
              1.4249322787e-1, -1.6668057665e-1, 2.0000714765e-1,
              -2.4999993993e-1, 3.3333331174e-1):
        p = p * xr + c
    y = xr * z * p
    y = y + e * -2.12194440e-4
    y = y - 0.5 * z
    return (xr + y) + e * 0.693359375


def _pass1_body(N, G, BPT, NX, XBASE,
                anch_h, gt_h, wh_h,
                lab_h, t0_h, t1_h, t2_h, t3_h, win_h, code_h, cnt_h,
                a_x0, a_y0, a_x1, a_y1,
                gt_v, g_ab, g_w, g_hh, g_cx, g_cy,
                lab_v, t0_v, t1_v, t2_v, t3_v, win_v, code_v,
                wh_v, cnt_v, sem):
    PT = BPT * L
    wid = lax.axis_index("s") * NC + lax.axis_index("c")
    base = wid * PT
    has_extra = wid < NX

    handles = []
    for p, dst in enumerate((a_x0, a_y0, a_x1, a_y1)):
        handles.append(pltpu.async_copy(anch_h.at[pl.ds(p * N + base, PT)],
                                        dst.at[pl.ds(0, PT)], sem))
    handles.append(pltpu.async_copy(gt_h, gt_v, sem))
    handles.append(pltpu.async_copy(wh_h, wh_v, sem))

    @pl.when(has_extra)
    def _():
        xoff = XBASE + wid * L
        for p, dst in enumerate((a_x0, a_y0, a_x1, a_y1)):
            pltpu.async_copy(anch_h.at[pl.ds(p * N + xoff, L)],
                             dst.at[pl.ds(PT, L)], sem).wait()
    for h in handles:
        h.wait()

    lanes = lax.iota(jnp.int32, 16)

    for c in range(G // L):
        sl = pl.ds(c * L, L)
        bx0 = gt_v[pl.ds(0 * G + c * L, L)]
        by0 = gt_v[pl.ds(1 * G + c * L, L)]
        bx1 = gt_v[pl.ds(2 * G + c * L, L)]
        by1 = gt_v[pl.ds(3 * G + c * L, L)]
        gw = bx1 - bx0
        gh = by1 - by0
        g_ab[sl] = gw * gh
        g_w[sl] = gw
        g_hh[sl] = gh
        g_cx[sl] = bx0 + 0.5 * gw
        g_cy[sl] = by0 + 0.5 * gh

    hv = wh_v[pl.ds(0, L)]
    wv = wh_v[pl.ds(L, L)]

    def chunk_body(i, acc):
        sl = pl.ds(i * L, L)
        ax0 = a_x0[sl]
        ay0 = a_y0[sl]
        ax1 = a_x1[sl]
        ay1 = a_y1[sl]
        aw = ax1 - ax0
        ah = ay1 - ay0
        area_a = aw * ah
        inside = ((ax0 >= 0.0) & (ay0 >= 0.0) & (ax1 <= wv) & (ay1 <= hv))

        def comb(a, b):
            u = b[0] > a[0]
            return (jnp.where(u, b[0], a[0]), jnp.where(u, b[1], a[1]))

        UNR = 2

        def gt_body(jb, carry):
            j0 = jb * UNR
            cands = []
            for k in range(UNR):
                jj = jnp.full((16,), j0 + k, jnp.int32)
                bx0 = plsc.load_gather(gt_v, [jj])
                by0 = plsc.load_gather(gt_v, [jj + G])
                bx1 = plsc.load_gather(gt_v, [jj + 2 * G])
                by1 = plsc.load_gather(gt_v, [jj + 3 * G])
                ab = plsc.load_gather(g_ab, [jj])
                wx = jnp.maximum(jnp.minimum(ax1, bx1) - jnp.maximum(ax0, bx0),
                                 0.0)
                wy = jnp.maximum(jnp.minimum(ay1, by1) - jnp.maximum(ay0, by0),
                                 0.0)
                inter = wx * wy
                iou = inter / ((area_a + ab) - inter)
                cands.append((iou, jj))
            while len(cands) > 1:
                cands = [comb(cands[k], cands[k + 1])
                         for k in range(0, len(cands), 2)]
            return comb(carry, cands[0])

        best_iou, best_idx = lax.fori_loop(
            0, G // UNR, gt_body,
            (jnp.full((16,), -1.0, jnp.float32), jnp.zeros((16,), jnp.int32)))

        neg = best_iou < NEG_IOU
        pos = best_iou >= POS_IOU
        labf = jnp.where(pos, 1.0, jnp.where(neg, 0.0, -1.0))
        lab_v[sl] = jnp.where(inside, labf, -1.0)

        bgx = plsc.load_gather(g_cx, [best_idx])
        bgy = plsc.load_gather(g_cy, [best_idx])
        bgw = plsc.load_gather(g_w, [best_idx])
        bgh = plsc.load_gather(g_hh, [best_idx])
        acx = ax0 + 0.5 * aw
        acy = ay0 + 0.5 * ah
        ones = jnp.full((16,), 1.0, jnp.float32)
        t0_v[sl] = jnp.where(inside, (bgx - acx) / aw, ones)
        t1_v[sl] = jnp.where(inside, (bgy - acy) / ah, ones)
        t2_v[sl] = jnp.where(inside, _vlog(bgw / aw), ones)
        t3_v[sl] = jnp.where(inside, _vlog(bgh / ah), ones)

        win_v[sl] = jnp.where(inside, jnp.where(pos, 1.0, 0.0), 1.0)
        code_v[sl] = jnp.where(inside, jnp.where(neg | pos, 2.0, 0.0), 1.0)

        validm = inside & (neg | pos)
        return acc + jnp.where(validm, 1.0, 0.0)

    nch = jnp.where(has_extra, BPT + 1, BPT)
    acc = lax.fori_loop(0, nch, chunk_body, jnp.zeros((16,), jnp.float32),
                        unroll=False)
    cnt_v[...] = acc

    outs = ((lab_v, lab_h), (t0_v, t0_h), (t1_v, t1_h), (t2_v, t2_h),
            (t3_v, t3_h), (win_v, win_h), (code_v, code_h))
    handles = []
    for src, dst in outs:
        handles.append(pltpu.async_copy(src.at[pl.ds(0, PT)],
                                        dst.at[pl.ds(base, PT)], sem))
    handles.append(pltpu.async_copy(cnt_v, cnt_h.at[pl.ds(wid * L, L)], sem))

    @pl.when(has_extra)
    def _():
        xb = XBASE + wid * L
        for src, dst in outs:
            pltpu.async_copy(src.at[pl.ds(PT, L)],
                             dst.at[pl.ds(xb, L)], sem).wait()
    for h in handles:
        h.wait()


def _wout_body(code_ref, cnt_ref, out_ref):
    inv = 1.0 / jnp.sum(cnt_ref[...])
    c = code_ref[...]
    out_ref[...] = jnp.where(c == 2.0, inv, jnp.where(c == 1.0, 1.0, 0.0))


@jax.jit
def kernel(gt_bboxes, image_shape, anchors):
    N = anchors.shape[0]
    G = gt_bboxes.shape[0]
    NV = N // L
    BPT = NV // NW
    NX = NV - BPT * NW
    XBASE = BPT * NW * L

    anch_t = anchors.T.reshape(-1)
    gt_t = gt_bboxes.T.reshape(-1)
    wh32 = jnp.repeat(image_shape, L)

    mesh = plsc.VectorSubcoreMesh(core_axis_name="c", subcore_axis_name="s",
                                  num_cores=NC, num_subcores=NS)
    cparams = pltpu.CompilerParams(needs_layout_passes=False)

    f32 = jnp.float32
    PT = BPT * L
    PTX = PT + L
    plane = jax.ShapeDtypeStruct((N,), f32)
    pass1 = pl.kernel(
        functools.partial(_pass1_body, N, G, BPT, NX, XBASE),
        out_type=(
            plane,
            plane, plane, plane, plane,
            plane,
            plane,
            jax.ShapeDtypeStruct((NW * L,), f32),
        ),
        mesh=mesh,
        compiler_params=cparams,
        scratch_types=(
            pltpu.VMEM((PTX,), f32), pltpu.VMEM((PTX,), f32),
            pltpu.VMEM((PTX,), f32), pltpu.VMEM((PTX,), f32),
            pltpu.VMEM((G * 4,), f32),
            pltpu.VMEM((G,), f32), pltpu.VMEM((G,), f32),
            pltpu.VMEM((G,), f32), pltpu.VMEM((G,), f32),
            pltpu.VMEM((G,), f32),
            pltpu.VMEM((PTX,), f32), pltpu.VMEM((PTX,), f32),
            pltpu.VMEM((PTX,), f32), pltpu.VMEM((PTX,), f32),
            pltpu.VMEM((PTX,), f32), pltpu.VMEM((PTX,), f32),
            pltpu.VMEM((PTX,), f32),
            pltpu.VMEM((2 * L,), f32),
            pltpu.VMEM((16,), f32),
            pltpu.SemaphoreType.DMA,
        ),
    )
    lab, t0, t1, t2, t3, winp, codep, counts = pass1(anch_t, gt_t, wh32)

    wop = pl.pallas_call(
        _wout_body,
        out_shape=jax.ShapeDtypeStruct((N,), f32),
    )(codep, counts)

    targets = jnp.stack([t0, t1, t2, t3], axis=1)
    w_in = jnp.broadcast_to(winp[:, None], (N, 4))
    w_out = jnp.broadcast_to(wop[:, None], (N, 4))
    return (lab, targets, w_in, w_out)


# speedup vs baseline: 1.0401x; 1.0401x over previous
import functools

import jax
import jax.numpy as jnp
from jax import lax
from jax.experimental import pallas as pl
from jax.experimental.pallas import tpu as pltpu
from jax.experimental.pallas import tpu_sc as plsc

NC = 2
NS = 16
L = 16
NW = NC * NS

POS_IOU = 0.7
NEG_IOU = 0.3


def _vlog(x):
    bits = plsc.bitcast(x, jnp.int32)
    e = ((bits >> 23) & 0xFF) - 127
    m = plsc.bitcast((bits & 0x007FFFFF) | 0x3F800000, jnp.float32)
    half = m * 0.5
    big = half >= 0.70710678118654752440
    xr = jnp.where(big, half, m) - 1.0
    e = (e + jnp.where(big, 1, 0)).astype(jnp.float32)
    z = xr * xr
    p = jnp.full_like(xr, 7.0376836292e-2)
    for c in (-1.1514610310e-1, 1.1676998740e-1, -1.2420140846e-1,
              1.4249322787e-1, -1.6668057665e-1, 2.0000714765e-1,
              -2.4999993993e-1, 3.3333331174e-1):
        p = p * xr + c
    y = xr * z * p
    y = y + e * -2.12194440e-4
    y = y - 0.5 * z
    return (xr + y) + e * 0.693359375


def _pass1_body(N, G, BPT, NX, XBASE,
                anch_h, gt_h, wh_h,
                lab_h, t0_h, t1_h, t2_h, t3_h, win_h, code_h, cnt_h,
                a_x0, a_y0, a_x1, a_y1,
                gt_v, g_ab, g_w, g_hh, g_cx, g_cy,
                lab_v, t0_v, t1_v, t2_v, t3_v, win_v, code_v,
                wh_v, cnt_v, sem):
    PT = BPT * L
    wid = lax.axis_index("s") * NC + lax.axis_index("c")
    base = wid * PT
    has_extra = wid < NX

    handles = []
    for p, dst in enumerate((a_x0, a_y0, a_x1, a_y1)):
        handles.append(pltpu.async_copy(anch_h.at[pl.ds(p * N + base, PT)],
                                        dst.at[pl.ds(0, PT)], sem))
    handles.append(pltpu.async_copy(gt_h, gt_v, sem))
    handles.append(pltpu.async_copy(wh_h, wh_v, sem))

    @pl.when(has_extra)
    def _():
        xoff = XBASE + wid * L
        for p, dst in enumerate((a_x0, a_y0, a_x1, a_y1)):
            pltpu.async_copy(anch_h.at[pl.ds(p * N + xoff, L)],
                             dst.at[pl.ds(PT, L)], sem).wait()
    for h in handles:
        h.wait()

    lanes = lax.iota(jnp.int32, 16)

    for c in range(G // L):
        sl = pl.ds(c * L, L)
        bx0 = gt_v[pl.ds(0 * G + c * L, L)]
        by0 = gt_v[pl.ds(1 * G + c * L, L)]
        bx1 = gt_v[pl.ds(2 * G + c * L, L)]
        by1 = gt_v[pl.ds(3 * G + c * L, L)]
        gw = bx1 - bx0
        gh = by1 - by0
        g_ab[sl] = gw * gh
        g_w[sl] = gw
        g_hh[sl] = gh
        g_cx[sl] = bx0 + 0.5 * gw
        g_cy[sl] = by0 + 0.5 * gh

    hv = wh_v[pl.ds(0, L)]
    wv = wh_v[pl.ds(L, L)]

    def chunk_body(i, acc):
        sl = pl.ds(i * L, L)
        ax0 = a_x0[sl]
        ay0 = a_y0[sl]
        ax1 = a_x1[sl]
        ay1 = a_y1[sl]
        aw = ax1 - ax0
        ah = ay1 - ay0
        area_a = aw * ah
        inside = ((ax0 >= 0.0) & (ay0 >= 0.0) & (ax1 <= wv) & (ay1 <= hv))

        def comb(a, b):
            u = b[0] > a[0]
            return (jnp.where(u, b[0], a[0]), jnp.where(u, b[1], a[1]))

        UNR = 4

        def gt_body(jb, carry):
            j0 = jb * UNR
            cands = []
            for k in range(UNR):
                jj = jnp.full((16,), j0 + k, jnp.int32)
                bx0 = plsc.load_gather(gt_v, [jj])
                by0 = plsc.load_gather(gt_v, [jj + G])
                bx1 = plsc.load_gather(gt_v, [jj + 2 * G])
                by1 = plsc.load_gather(gt_v, [jj + 3 * G])
                ab = plsc.load_gather(g_ab, [jj])
                wx = jnp.maximum(jnp.minimum(ax1, bx1) - jnp.maximum(ax0, bx0),
                                 0.0)
                wy = jnp.maximum(jnp.minimum(ay1, by1) - jnp.maximum(ay0, by0),
                                 0.0)
                inter = wx * wy
                iou = inter / ((area_a + ab) - inter)
                cands.append((iou, jj))
            while len(cands) > 1:
                cands = [comb(cands[k], cands[k + 1])
                         for k in range(0, len(cands), 2)]
            return comb(carry, cands[0])

        best_iou, best_idx = lax.fori_loop(
            0, G // UNR, gt_body,
            (jnp.full((16,), -1.0, jnp.float32), jnp.zeros((16,), jnp.int32)))

        neg = best_iou < NEG_IOU
        pos = best_iou >= POS_IOU
        labf = jnp.where(pos, 1.0, jnp.where(neg, 0.0, -1.0))
        lab_v[sl] = jnp.where(inside, labf, -1.0)

        bgx = plsc.load_gather(g_cx, [best_idx])
        bgy = plsc.load_gather(g_cy, [best_idx])
        bgw = plsc.load_gather(g_w, [best_idx])
        bgh = plsc.load_gather(g_hh, [best_idx])
        acx = ax0 + 0.5 * aw
        acy = ay0 + 0.5 * ah
        ones = jnp.full((16,), 1.0, jnp.float32)
        t0_v[sl] = jnp.where(inside, (bgx - acx) / aw, ones)
        t1_v[sl] = jnp.where(inside, (bgy - acy) / ah, ones)
        t2_v[sl] = jnp.where(inside, _vlog(bgw / aw), ones)
        t3_v[sl] = jnp.where(inside, _vlog(bgh / ah), ones)

        win_v[sl] = jnp.where(inside, jnp.where(pos, 1.0, 0.0), 1.0)
        code_v[sl] = jnp.where(inside, jnp.where(neg | pos, 2.0, 0.0), 1.0)

        validm = inside & (neg | pos)
        return acc + jnp.where(validm, 1.0, 0.0)

    nch = jnp.where(has_extra, BPT + 1, BPT)
    acc = lax.fori_loop(0, nch, chunk_body, jnp.zeros((16,), jnp.float32),
                        unroll=False)
    cnt_v[...] = acc

    outs = ((lab_v, lab_h), (t0_v, t0_h), (t1_v, t1_h), (t2_v, t2_h),
            (t3_v, t3_h), (win_v, win_h), (code_v, code_h))
    handles = []
    for src, dst in outs:
        handles.append(pltpu.async_copy(src.at[pl.ds(0, PT)],
                                        dst.at[pl.ds(base, PT)], sem))
    handles.append(pltpu.async_copy(cnt_v, cnt_h.at[pl.ds(wid * L, L)], sem))

    @pl.when(has_extra)
    def _():
        xb = XBASE + wid * L
        for src, dst in outs:
            pltpu.async_copy(src.at[pl.ds(PT, L)],
                             dst.at[pl.ds(xb, L)], sem).wait()
    for h in handles:
        h.wait()


def _wout_body(code_ref, cnt_ref, out_ref):
    inv = 1.0 / jnp.sum(cnt_ref[...])
    c = code_ref[...]
    out_ref[...] = jnp.where(c == 2.0, inv, jnp.where(c == 1.0, 1.0, 0.0))


@jax.jit
def kernel(gt_bboxes, image_shape, anchors):
    N = anchors.shape[0]
    G = gt_bboxes.shape[0]
    NV = N // L
    BPT = NV // NW
    NX = NV - BPT * NW
    XBASE = BPT * NW * L

    anch_t = anchors.T.reshape(-1)
    gt_t = gt_bboxes.T.reshape(-1)
    wh32 = jnp.repeat(image_shape, L)

    mesh = plsc.VectorSubcoreMesh(core_axis_name="c", subcore_axis_name="s",
                                  num_cores=NC, num_subcores=NS)
    cparams = pltpu.CompilerParams(needs_layout_passes=False)

    f32 = jnp.float32
    PT = BPT * L
    PTX = PT + L
    plane = jax.ShapeDtypeStruct((N,), f32)
    pass1 = pl.kernel(
        functools.partial(_pass1_body, N, G, BPT, NX, XBASE),
        out_type=(
            plane,
            plane, plane, plane, plane,
            plane,
            plane,
            jax.ShapeDtypeStruct((NW * L,), f32),
        ),
        mesh=mesh,
        compiler_params=cparams,
        scratch_types=(
            pltpu.VMEM((PTX,), f32), pltpu.VMEM((PTX,), f32),
            pltpu.VMEM((PTX,), f32), pltpu.VMEM((PTX,), f32),
            pltpu.VMEM((G * 4,), f32),
            pltpu.VMEM((G,), f32), pltpu.VMEM((G,), f32),
            pltpu.VMEM((G,), f32), pltpu.VMEM((G,), f32),
            pltpu.VMEM((G,), f32),
            pltpu.VMEM((PTX,), f32), pltpu.VMEM((PTX,), f32),
            pltpu.VMEM((PTX,), f32), pltpu.VMEM((PTX,), f32),
            pltpu.VMEM((PTX,), f32), pltpu.VMEM((PTX,), f32),
            pltpu.VMEM((PTX,), f32),
            pltpu.VMEM((2 * L,), f32),
            pltpu.VMEM((16,), f32),
            pltpu.SemaphoreType.DMA,
        ),
    )
    lab, t0, t1, t2, t3, winp, codep, counts = pass1(anch_t, gt_t, wh32)

    wop = pl.pallas_call(
        _wout_body,
        out_shape=jax.ShapeDtypeStruct((N,), f32),
    )(codep, counts)

    targets = jnp.stack([t0, t1, t2, t3], axis=1)
    w_in = jnp.broadcast_to(winp[:, None], (N, 4))
    w_out = jnp.broadcast_to(wop[:, None], (N, 4))
    return (lab, targets, w_in, w_out)
